# Initial kernel scaffold; baseline (speedup 1.0000x reference)
import jax
import jax.numpy as jnp
from jax.experimental import pallas as pl


def kernel(x, q):
    raise NotImplementedError("write your pallas kernel here")


# speedup vs baseline: 3.3977x; 3.3977x over previous
import jax
import jax.numpy as jnp
from jax.experimental import pallas as pl
from jax.experimental.pallas import tpu as pltpu

D = 4096
G = 128
NG = D // G
EPS = 1e-8
BR = 128


def _coef_vectors(q):
    w, x, y, z = q[:, 0], q[:, 1], q[:, 2], q[:, 3]
    o = jnp.zeros_like(w)

    def inter(a, b, c, d):
        return jnp.stack([a, b, c, d], axis=-1).reshape(-1)

    c0 = inter(w, w, w, w)
    cp1 = inter(-x, -z, -x, o)
    cm1 = inter(o, x, z, x)
    cp2 = inter(-y, y, o, o)
    cm2 = inter(o, o, y, -y)
    cp3 = inter(-z, o, o, o)
    cm3 = inter(o, o, o, z)
    fwd = jnp.stack([c0, cp1, cm1, cp2, cm2, cp3, cm3])
    inv = jnp.concatenate([c0[None], -fwd[1:]], axis=0)
    return jnp.concatenate([fwd, inv], axis=0).astype(jnp.float32)


def _rotate(v, cf, base):
    def c(i):
        return cf[base + i:base + i + 1, :]

    r = v * c(0)
    r = r + pltpu.roll(v, D - 1, axis=1) * c(1)
    r = r + pltpu.roll(v, 1, axis=1) * c(2)
    r = r + pltpu.roll(v, D - 2, axis=1) * c(3)
    r = r + pltpu.roll(v, 2, axis=1) * c(4)
    r = r + pltpu.roll(v, D - 3, axis=1) * c(5)
    r = r + pltpu.roll(v, 3, axis=1) * c(6)
    return r


def _body(x_ref, cf_ref, m_ref, mt_ref, o_ref, s_ref, n_ref):
    x = x_ref[...]
    cf = cf_ref[...]
    n2 = jnp.sum(x * x, axis=1, keepdims=True)
    n = jnp.maximum(jnp.sqrt(n2), EPS)
    xr = _rotate(x, cf, 0)
    ssum = jnp.dot(jnp.abs(xr), m_ref[...],
                   preferred_element_type=jnp.float32)
    scales = ssum * ((1.0 / G) / n)
    factor = jnp.maximum(ssum * (1.0 / G), EPS * n)
    b = jnp.dot(factor, mt_ref[...],
                preferred_element_type=jnp.float32)
    mid = jnp.where(xr < 0, -b, b)
    o_ref[...] = _rotate(mid, cf, 7)
    s_ref[...] = scales
    n_ref[...] = n


def kernel(x, q):
    n_tokens = x.shape[0]
    cf = _coef_vectors(q)
    m = jnp.repeat(jnp.eye(NG, dtype=jnp.float32), G, axis=0)
    mt = m.T
    grid = (n_tokens // BR,)
    x_hat, scales, norms = pl.pallas_call(
        _body,
        grid=grid,
        in_specs=[
            pl.BlockSpec((BR, D), lambda i: (i, 0)),
            pl.BlockSpec((14, D), lambda i: (0, 0)),
            pl.BlockSpec((D, NG), lambda i: (0, 0)),
            pl.BlockSpec((NG, D), lambda i: (0, 0)),
        ],
        out_specs=[
            pl.BlockSpec((BR, D), lambda i: (i, 0)),
            pl.BlockSpec((BR, NG), lambda i: (i, 0)),
            pl.BlockSpec((BR, 1), lambda i: (i, 0)),
        ],
        out_shape=[
            jax.ShapeDtypeStruct((n_tokens, D), jnp.float32),
            jax.ShapeDtypeStruct((n_tokens, NG), jnp.float32),
            jax.ShapeDtypeStruct((n_tokens, 1), jnp.float32),
        ],
        compiler_params=pltpu.CompilerParams(
            dimension_semantics=("parallel",),
        ),
    )(x, cf, m, mt)
    return (x_hat, scales, norms[:, 0])


# speedup vs baseline: 5.8129x; 1.7108x over previous
import jax
import jax.numpy as jnp
from jax.experimental import pallas as pl
from jax.experimental.pallas import tpu as pltpu

D = 4096
G = 128
NG = D // G
NC = 16
CW = D // NC
EPS = 1e-8
BR = 128


def _coef_vectors(q):
    w, x, y, z = q[:, 0], q[:, 1], q[:, 2], q[:, 3]
    o = jnp.zeros_like(w)

    def inter(a, b, c, d):
        return jnp.stack([a, b, c, d], axis=-1).reshape(-1)

    c0 = inter(w, w, w, w)
    cp1 = inter(-x, -z, -x, o)
    cm1 = inter(o, x, z, x)
    cp2 = inter(-y, y, o, o)
    cm2 = inter(o, o, y, -y)
    cp3 = inter(-z, o, o, o)
    cm3 = inter(o, o, o, z)
    return jnp.stack([c0, cp1, cm1, cp2, cm2, cp3, cm3]).astype(jnp.float32)


def _rot_mats(q):
    w, x, y, z = q[:, 0], q[:, 1], q[:, 2], q[:, 3]
    f = jnp.stack([
        jnp.stack([w, -x, -y, -z], axis=-1),
        jnp.stack([x, w, -z, y], axis=-1),
        jnp.stack([y, z, w, -x], axis=-1),
        jnp.stack([z, -y, x, w], axis=-1),
    ], axis=-2)
    ft = jnp.transpose(f, (0, 2, 1))
    ft = ft.reshape(NC, CW // 4, 4, 4)
    eye = jnp.eye(CW // 4, dtype=q.dtype)
    r = eye.reshape(1, CW // 4, 1, CW // 4, 1) * ft.reshape(NC, CW // 4, 4, 1, 4)
    return r.reshape(NC, CW, CW)


def _rotate_fwd(v, cf):
    def c(i):
        return cf[i:i + 1, :]

    r = v * c(0)
    r = r + pltpu.roll(v, D - 1, axis=1) * c(1)
    r = r + pltpu.roll(v, 1, axis=1) * c(2)
    r = r + pltpu.roll(v, D - 2, axis=1) * c(3)
    r = r + pltpu.roll(v, 2, axis=1) * c(4)
    r = r + pltpu.roll(v, D - 3, axis=1) * c(5)
    r = r + pltpu.roll(v, 3, axis=1) * c(6)
    return r


def _body(x_ref, cf_ref, rinv_ref, m_ref, mt_ref, o_ref, s_ref, n_ref):
    x = x_ref[...]
    n2 = jnp.sum(x * x, axis=1, keepdims=True)
    n = jnp.maximum(jnp.sqrt(n2), EPS)

    xr = _rotate_fwd(x, cf_ref[...])

    ab = jnp.abs(xr).astype(jnp.bfloat16)
    ssum = jnp.dot(ab, m_ref[...], preferred_element_type=jnp.float32)
    scales = ssum * ((1.0 / G) / n)
    factor = jnp.maximum(ssum * (1.0 / G), EPS * n)
    b = jnp.dot(factor.astype(jnp.bfloat16), mt_ref[...],
                preferred_element_type=jnp.float32)

    sgn = jnp.where(xr < 0, -1.0, 1.0).astype(jnp.bfloat16)
    for t in range(NC):
        sl = slice(t * CW, (t + 1) * CW)
        u_t = jnp.dot(sgn[:, sl], rinv_ref[t], preferred_element_type=jnp.float32)
        o_ref[:, sl] = u_t * b[:, sl]
    s_ref[...] = scales
    n_ref[...] = n


def kernel(x, q):
    n_tokens = x.shape[0]
    qf = q.astype(jnp.float32)
    cf = _coef_vectors(qf)
    qc = qf * jnp.array([1.0, -1.0, -1.0, -1.0], dtype=jnp.float32)
    r_inv = _rot_mats(qc).astype(jnp.bfloat16)
    m = jnp.repeat(jnp.eye(NG, dtype=jnp.bfloat16), G, axis=0)
    mt = m.T
    grid = (n_tokens // BR,)
    x_hat, scales, norms = pl.pallas_call(
        _body,
        grid=grid,
        in_specs=[
            pl.BlockSpec((BR, D), lambda i: (i, 0)),
            pl.BlockSpec((7, D), lambda i: (0, 0)),
            pl.BlockSpec((NC, CW, CW), lambda i: (0, 0, 0)),
            pl.BlockSpec((D, NG), lambda i: (0, 0)),
            pl.BlockSpec((NG, D), lambda i: (0, 0)),
        ],
        out_specs=[
            pl.BlockSpec((BR, D), lambda i: (i, 0)),
            pl.BlockSpec((BR, NG), lambda i: (i, 0)),
            pl.BlockSpec((BR, 1), lambda i: (i, 0)),
        ],
        out_shape=[
            jax.ShapeDtypeStruct((n_tokens, D), jnp.float32),
            jax.ShapeDtypeStruct((n_tokens, NG), jnp.float32),
            jax.ShapeDtypeStruct((n_tokens, 1), jnp.float32),
        ],
        compiler_params=pltpu.CompilerParams(
            dimension_semantics=("parallel",),
        ),
    )(x, cf, r_inv, m, mt)
    return (x_hat, scales, norms[:, 0])
